# baseline (speedup 1.0000x reference)
import jax
import jax.numpy as jnp
from jax import lax
from jax.experimental import pallas as pl
from jax.experimental.pallas import tpu as pltpu
from jax.experimental.pallas import tpu_sc as plsc

N = 10000
NP = 10240
E = 320000
D = 128
NC = 2
NS = 16
WIN = 512
EPAD = 327680
WPT = EPAD // (NC * NS * WIN)
ROWS_PT = NP // NS
CW = 4
NCHUNK = WPT // CW
BN = 256
DEG8 = 8
DWIN = 8
BF = jnp.bfloat16


def _mesh():
    return plsc.VectorSubcoreMesh(core_axis_name="c", subcore_axis_name="s",
                                  num_cores=NC, num_subcores=NS)


_SC_PARAMS = pltpu.CompilerParams(use_tc_tiling_on_sc=False)


def _deg_body(idx4, out, idxb, ones, zwin, sem, acc):
    c = lax.axis_index("c")
    s = lax.axis_index("s")

    def of(i, _):
        ones[pl.ds(i * 16, 16)] = jnp.ones((16,), jnp.float32)
        return _
    lax.fori_loop(0, DWIN * WIN // 16, of, 0)

    def zf(i, _):
        zwin[pl.ds(i * 16, 16)] = jnp.zeros((16,), jnp.float32)
        return _
    lax.fori_loop(0, 80, zf, 0)
    zchunk = NP * DEG8 // NS
    zbase = s * zchunk
    for k in range(zchunk // 1280):
        pltpu.sync_copy(zwin, acc.at[pl.ds(zbase + k * 1280, 1280)])
    plsc.subcore_barrier()

    bw = DWIN * WIN
    epw = EPAD // (NC * NS)
    eb = (c * NS + s) * epw
    nbig = epw // bw
    for a in range(4):
        pltpu.sync_copy(idx4.at[a, pl.ds(eb, epw)], idxb.at[a])
        for w in range(nbig):
            pltpu.async_copy(
                ones, acc.at[idxb.at[a, pl.ds(w * bw, bw)]],
                sem, add=True)
    for a in range(4):
        for w in range(nbig):
            pltpu.make_async_copy(
                ones, acc.at[idxb.at[0, pl.ds(0, bw)]], sem).wait()

    plsc.subcore_barrier()
    pltpu.sync_copy(acc.at[pl.ds(zbase, zchunk)],
                    out.at[c, pl.ds(zbase, zchunk)])


def _degrees(idx4):
    f = pl.kernel(
        _deg_body,
        out_type=jax.ShapeDtypeStruct((NC, NP * DEG8), jnp.float32),
        mesh=_mesh(),
        scratch_types=[
            pltpu.VMEM((4, EPAD // (NC * NS)), jnp.int32),
            pltpu.VMEM((DWIN * WIN,), jnp.float32),
            pltpu.VMEM((1280,), jnp.float32),
            pltpu.SemaphoreType.DMA,
            pltpu.VMEM_SHARED((NP * DEG8,), jnp.float32),
        ],
        compiler_params=_SC_PARAMS,
    )
    return f(idx4)


def _spmm_body(tbl, src2d, dst2d, out, srcc0, srcc1, dstc0, dstc1,
               rows0, rows1, semg0, semg1, semi, acc):
    c = lax.axis_index("c")
    s = lax.axis_index("s")

    def zrow(i, _):
        for j in range(D // 32):
            rows0[i, pl.ds(j * 32, 32)] = jnp.zeros((32,), BF)
        return _
    lax.fori_loop(0, WIN, zrow, 0)
    r0 = s * ROWS_PT
    zch = 128
    for k in range(ROWS_PT // zch):
        pltpu.sync_copy(rows0.at[pl.ds(0, zch)],
                        acc.at[pl.ds(r0 + k * zch, zch)])
    plsc.subcore_barrier()

    eb = (c * NS + s) * WPT
    srccs = (srcc0, srcc1)
    dstcs = (dstc0, dstc1)

    def load_idx_async(k, kb):
        pltpu.async_copy(src2d.at[pl.ds(eb + k * CW, CW)], srccs[kb], semi)
        pltpu.async_copy(dst2d.at[pl.ds(eb + k * CW, CW)], dstcs[kb], semi)

    def wait_idx(kb):
        pltpu.make_async_copy(src2d.at[pl.ds(eb, CW)], srccs[kb], semi).wait()
        pltpu.make_async_copy(dst2d.at[pl.ds(eb, CW)], dstcs[kb], semi).wait()

    pltpu.sync_copy(src2d.at[pl.ds(eb, CW)], srcc0)
    pltpu.sync_copy(dst2d.at[pl.ds(eb, CW)], dstc0)
    load_idx_async(1, 1)

    pltpu.async_copy(tbl.at[srcc0.at[0]], rows0, semg0)
    pltpu.async_copy(tbl.at[srcc0.at[1]], rows1, semg1)

    bufs = ((rows0, semg0), (rows1, semg1))

    def chunk(k, _):
        for b in range(CW):
            rows, semg = bufs[b % 2]
            for kb in range(2):
                @pl.when(k % 2 == kb)
                def _do():
                    pltpu.make_async_copy(
                        tbl.at[srccs[kb].at[b]], rows, semg).wait()
                    pltpu.sync_copy(rows, acc.at[dstcs[kb].at[b]], add=True)
                    if b == CW - 2:
                        @pl.when(k + 1 < NCHUNK)
                        def _w():
                            wait_idx(1 - kb)
                    if b + 2 < CW:
                        pltpu.async_copy(tbl.at[srccs[kb].at[b + 2]], rows,
                                         semg)
                    else:
                        @pl.when(k + 1 < NCHUNK)
                        def _x():
                            pltpu.async_copy(
                                tbl.at[srccs[1 - kb].at[b + 2 - CW]], rows,
                                semg)

        @pl.when(k + 2 < NCHUNK)
        def _pf():
            for kb in range(2):
                @pl.when(k % 2 == kb)
                def _pf2():
                    load_idx_async(k + 2, kb)
        return _
    lax.fori_loop(0, NCHUNK, chunk, 0)

    plsc.subcore_barrier()
    pltpu.sync_copy(acc.at[pl.ds(r0, ROWS_PT)], out.at[c, pl.ds(r0, ROWS_PT)])


def _spmm(tbl, src2d, dst2d):
    f = pl.kernel(
        _spmm_body,
        out_type=jax.ShapeDtypeStruct((NC, NP, D), BF),
        mesh=_mesh(),
        scratch_types=[
            pltpu.VMEM((CW, WIN), jnp.int32),
            pltpu.VMEM((CW, WIN), jnp.int32),
            pltpu.VMEM((CW, WIN), jnp.int32),
            pltpu.VMEM((CW, WIN), jnp.int32),
            pltpu.VMEM((WIN, D), BF),
            pltpu.VMEM((WIN, D), BF),
            pltpu.SemaphoreType.DMA,
            pltpu.SemaphoreType.DMA,
            pltpu.SemaphoreType.DMA,
            pltpu.VMEM_SHARED((NP, D), BF),
        ],
        compiler_params=_SC_PARAMS,
    )
    return f(tbl, src2d, dst2d)


def _prep_body(cnt_ref, f1_ref, f2_ref, sc1_ref, sc2_ref, inv_ref):
    cnt = cnt_ref[0] + cnt_ref[1]
    inv = lax.rsqrt(jnp.maximum(cnt, 1.0))
    inv_ref[...] = inv
    sc1_ref[...] = (f1_ref[...] * inv[:, 0:1]).astype(BF)
    sc2_ref[...] = (f2_ref[...] * inv[:, 2:3]).astype(BF)


def _prep(cnts, f1p, f2p):
    grid = NP // BN
    return pl.pallas_call(
        _prep_body,
        grid=(grid,),
        in_specs=[
            pl.BlockSpec((NC, BN, DEG8), lambda i: (0, i, 0)),
            pl.BlockSpec((BN, D), lambda i: (i, 0)),
            pl.BlockSpec((BN, D), lambda i: (i, 0)),
        ],
        out_specs=[
            pl.BlockSpec((BN, D), lambda i: (i, 0)),
            pl.BlockSpec((BN, D), lambda i: (i, 0)),
            pl.BlockSpec((BN, DEG8), lambda i: (i, 0)),
        ],
        out_shape=[
            jax.ShapeDtypeStruct((NP, D), BF),
            jax.ShapeDtypeStruct((NP, D), BF),
            jax.ShapeDtypeStruct((NP, DEG8), jnp.float32),
        ],
    )(cnts, f1p, f2p)


def _mid_body(s1_ref, s2_ref, inv_ref, w11_ref, b11_ref, w21_ref, b21_ref,
              w12_ref, w22_ref, y1a_ref, y1b_ref, y2a_ref, y2b_ref):
    inv = inv_ref[...]
    for g, (s_ref, ya_ref, yb_ref) in enumerate(
            ((s1_ref, y1a_ref, y1b_ref), (s2_ref, y2a_ref, y2b_ref))):
        inv_in = inv[:, 2 * g + 1:2 * g + 2]
        inv_out = inv[:, 2 * g:2 * g + 1]
        S = (s_ref[0].astype(jnp.float32)
             + s_ref[1].astype(jnp.float32)) * inv_in
        for (w1_ref, b1_ref, w2_ref, y_ref) in (
                (w11_ref, b11_ref, w12_ref, ya_ref),
                (w21_ref, b21_ref, w22_ref, yb_ref)):
            X = jnp.maximum(
                jnp.dot(S, w1_ref[...],
                        preferred_element_type=jnp.float32) + b1_ref[...],
                0.0)
            y_ref[...] = (jnp.dot(
                X, w2_ref[...],
                preferred_element_type=jnp.float32) * inv_out).astype(BF)


def _mid(S1, S2, invs, W11, b11, W21, b21, W12, W22):
    grid = NP // BN
    full = lambda shape: pl.BlockSpec(shape, lambda i: tuple(0 for _ in shape))
    return pl.pallas_call(
        _mid_body,
        grid=(grid,),
        in_specs=[
            pl.BlockSpec((NC, BN, D), lambda i: (0, i, 0)),
            pl.BlockSpec((NC, BN, D), lambda i: (0, i, 0)),
            pl.BlockSpec((BN, DEG8), lambda i: (i, 0)),
            full((D, 2 * D)), full((1, 2 * D)),
            full((D, 2 * D)), full((1, 2 * D)),
            full((2 * D, D)), full((2 * D, D)),
        ],
        out_specs=[pl.BlockSpec((BN, D), lambda i: (i, 0))] * 4,
        out_shape=[jax.ShapeDtypeStruct((NP, D), BF)] * 4,
    )(S1, S2, invs, W11, b11, W21, b21, W12, W22)


def _fin_body(t1a_ref, t1b_ref, t2a_ref, t2b_ref, inv_ref,
              b12_ref, b22_ref, p11w_ref, p11b_ref, p12w_ref, p12b_ref,
              p21w_ref, p21b_ref, p22w_ref, p22b_ref,
              z1_ref, z2_ref, z1b_ref, z2b_ref):
    inv = inv_ref[...]
    jobs = (
        (t1a_ref, 0, b12_ref, p11w_ref, p11b_ref, p12w_ref, p12b_ref, z1_ref),
        (t2a_ref, 1, b12_ref, p11w_ref, p11b_ref, p12w_ref, p12b_ref, z2_ref),
        (t1b_ref, 0, b22_ref, p21w_ref, p21b_ref, p22w_ref, p22b_ref, z1b_ref),
        (t2b_ref, 1, b22_ref, p21w_ref, p21b_ref, p22w_ref, p22b_ref, z2b_ref),
    )
    for (t_ref, g, b2_ref, p1w_ref, p1b_ref, p2w_ref, p2b_ref, z_ref) in jobs:
        x = jnp.maximum(
            (t_ref[0].astype(jnp.float32) + t_ref[1].astype(jnp.float32))
            * inv[:, 2 * g + 1:2 * g + 2] + b2_ref[...],
            0.0)
        u = jnp.dot(x, p1w_ref[...],
                    preferred_element_type=jnp.float32) + p1b_ref[...]
        u = jnp.where(u > 0.0, u, jnp.exp(u) - 1.0)
        z_ref[...] = jnp.dot(u, p2w_ref[...],
                             preferred_element_type=jnp.float32) + p2b_ref[...]


def _fin(T1a, T1b, T2a, T2b, invs, b12, b22,
         p11w, p11b, p12w, p12b, p21w, p21b, p22w, p22b):
    grid = NP // BN
    full = lambda shape: pl.BlockSpec(shape, lambda i: tuple(0 for _ in shape))
    return pl.pallas_call(
        _fin_body,
        grid=(grid,),
        in_specs=[
            pl.BlockSpec((NC, BN, D), lambda i: (0, i, 0)),
            pl.BlockSpec((NC, BN, D), lambda i: (0, i, 0)),
            pl.BlockSpec((NC, BN, D), lambda i: (0, i, 0)),
            pl.BlockSpec((NC, BN, D), lambda i: (0, i, 0)),
            pl.BlockSpec((BN, DEG8), lambda i: (i, 0)),
            full((1, D)), full((1, D)),
            full((D, D)), full((1, D)), full((D, D)), full((1, D)),
            full((D, D)), full((1, D)), full((D, D)), full((1, D)),
        ],
        out_specs=[pl.BlockSpec((BN, D), lambda i: (i, 0))] * 4,
        out_shape=[jax.ShapeDtypeStruct((NP, D), jnp.float32)] * 4,
    )(T1a, T1b, T2a, T2b, invs, b12, b22,
      p11w, p11b, p12w, p12b, p21w, p21b, p22w, p22b)


def kernel(feat1, feat2, edge_index1, edge_index2,
           e1_W1, e1_b1, e1_W2, e1_b2, e1_p1W, e1_p1b, e1_p2W, e1_p2b,
           e2_W1, e2_b1, e2_W2, e2_b2, e2_p1W, e2_p1b, e2_p2W, e2_p2b,
           epoch, threshold, split_size):
    del epoch, threshold, split_size

    npad = EPAD - E
    padv = (jnp.arange(npad, dtype=jnp.int32) % (NP - N)) + N

    def pad_idx(x):
        return jnp.concatenate([x.astype(jnp.int32), padv])

    s1f = pad_idx(edge_index1[0])
    d1f = pad_idx(edge_index1[1])
    s2f = pad_idx(edge_index2[0])
    d2f = pad_idx(edge_index2[1])
    s1 = s1f.reshape(EPAD // WIN, WIN)
    d1 = d1f.reshape(EPAD // WIN, WIN)
    s2 = s2f.reshape(EPAD // WIN, WIN)
    d2 = d2f.reshape(EPAD // WIN, WIN)

    idx4 = jnp.stack([s1f * DEG8, d1f * DEG8 + 1,
                      s2f * DEG8 + 2, d2f * DEG8 + 3])
    cnts = _degrees(idx4).reshape(NC, NP, DEG8)

    f1p = jnp.pad(feat1, ((0, NP - N), (0, 0)))
    f2p = jnp.pad(feat2, ((0, NP - N), (0, 0)))
    featsc1, featsc2, invs = _prep(cnts, f1p, f2p)

    S1 = _spmm(featsc1, s1, d1)
    S2 = _spmm(featsc2, s2, d2)

    y1a, y1b, y2a, y2b = _mid(
        S1, S2, invs,
        e1_W1, e1_b1.reshape(1, -1), e2_W1, e2_b1.reshape(1, -1),
        e1_W2, e2_W2)

    T1a = _spmm(y1a, s1, d1)
    T1b = _spmm(y1b, s1, d1)
    T2a = _spmm(y2a, s2, d2)
    T2b = _spmm(y2b, s2, d2)

    z1, z2, z1_, z2_ = _fin(
        T1a, T1b, T2a, T2b, invs,
        e1_b2.reshape(1, -1), e2_b2.reshape(1, -1),
        e1_p1W, e1_p1b.reshape(1, -1), e1_p2W, e1_p2b.reshape(1, -1),
        e2_p1W, e2_p1b.reshape(1, -1), e2_p2W, e2_p2b.reshape(1, -1))

    return (z1[:N], z2[:N], z1_[:N], z2_[:N])


# speedup vs baseline: 1.0302x; 1.0302x over previous
import jax
import jax.numpy as jnp
from jax import lax
from jax.experimental import pallas as pl
from jax.experimental.pallas import tpu as pltpu
from jax.experimental.pallas import tpu_sc as plsc

N = 10000
NP = 10240
E = 320000
D = 128
NC = 2
NS = 16
WIN = 256
EPAD = 327680
WPT = EPAD // (NC * NS * WIN)
ROWS_PT = NP // NS
CW = 8
NCHUNK = WPT // CW
BN = 256
DEG8 = 8
DBW = 2048
BF = jnp.bfloat16


def _mesh():
    return plsc.VectorSubcoreMesh(core_axis_name="c", subcore_axis_name="s",
                                  num_cores=NC, num_subcores=NS)


_SC_PARAMS = pltpu.CompilerParams(use_tc_tiling_on_sc=False)


def _deg_body(idx4, out, idxb, ones, zwin, sem, acc):
    c = lax.axis_index("c")
    s = lax.axis_index("s")

    def of(i, _):
        ones[pl.ds(i * 16, 16)] = jnp.ones((16,), jnp.float32)
        return _
    lax.fori_loop(0, DBW // 16, of, 0)

    def zf(i, _):
        zwin[pl.ds(i * 16, 16)] = jnp.zeros((16,), jnp.float32)
        return _
    lax.fori_loop(0, 80, zf, 0)
    zchunk = NP * DEG8 // NS
    zbase = s * zchunk
    for k in range(zchunk // 1280):
        pltpu.sync_copy(zwin, acc.at[pl.ds(zbase + k * 1280, 1280)])
    plsc.subcore_barrier()

    bw = DBW
    epw = EPAD // (NC * NS)
    eb = (c * NS + s) * epw
    nbig = epw // bw
    for a in range(4):
        pltpu.sync_copy(idx4.at[a, pl.ds(eb, epw)], idxb.at[a])
        for w in range(nbig):
            pltpu.async_copy(
                ones, acc.at[idxb.at[a, pl.ds(w * bw, bw)]],
                sem, add=True)
    for a in range(4):
        for w in range(nbig):
            pltpu.make_async_copy(
                ones, acc.at[idxb.at[0, pl.ds(0, bw)]], sem).wait()

    plsc.subcore_barrier()
    pltpu.sync_copy(acc.at[pl.ds(zbase, zchunk)],
                    out.at[c, pl.ds(zbase, zchunk)])


def _degrees(idx4):
    f = pl.kernel(
        _deg_body,
        out_type=jax.ShapeDtypeStruct((NC, NP * DEG8), jnp.float32),
        mesh=_mesh(),
        scratch_types=[
            pltpu.VMEM((4, EPAD // (NC * NS)), jnp.int32),
            pltpu.VMEM((DBW,), jnp.float32),
            pltpu.VMEM((1280,), jnp.float32),
            pltpu.SemaphoreType.DMA,
            pltpu.VMEM_SHARED((NP * DEG8,), jnp.float32),
        ],
        compiler_params=_SC_PARAMS,
    )
    return f(idx4)


def _make_spmm(win, tw, cw):
    wpt = EPAD // (NC * NS * win)
    nchunk = wpt // cw

    def body(tbl, src2d, dst2d, out, srcc0, srcc1, dstc0, dstc1,
             rows0, rows1, semg0, semg1, semi, acc):
        c = lax.axis_index("c")
        s = lax.axis_index("s")

        def zrow(i, _):
            for j in range(tw // 32):
                rows0[i, pl.ds(j * 32, 32)] = jnp.zeros((32,), BF)
            return _
        lax.fori_loop(0, win, zrow, 0)
        r0 = s * ROWS_PT
        zch = min(win, 128)
        for k in range(ROWS_PT // zch):
            pltpu.sync_copy(rows0.at[pl.ds(0, zch)],
                            acc.at[pl.ds(r0 + k * zch, zch)])
        plsc.subcore_barrier()

        eb = (c * NS + s) * wpt
        srccs = (srcc0, srcc1)
        dstcs = (dstc0, dstc1)

        def load_idx_async(k, kb):
            pltpu.async_copy(src2d.at[pl.ds(eb + k * cw, cw)], srccs[kb],
                             semi)
            pltpu.async_copy(dst2d.at[pl.ds(eb + k * cw, cw)], dstcs[kb],
                             semi)

        def wait_idx(kb):
            pltpu.make_async_copy(src2d.at[pl.ds(eb, cw)], srccs[kb],
                                  semi).wait()
            pltpu.make_async_copy(dst2d.at[pl.ds(eb, cw)], dstcs[kb],
                                  semi).wait()

        pltpu.sync_copy(src2d.at[pl.ds(eb, cw)], srcc0)
        pltpu.sync_copy(dst2d.at[pl.ds(eb, cw)], dstc0)
        load_idx_async(1, 1)

        pltpu.async_copy(tbl.at[srcc0.at[0]], rows0, semg0)
        pltpu.async_copy(tbl.at[srcc0.at[1]], rows1, semg1)

        bufs = ((rows0, semg0), (rows1, semg1))

        def chunk(k, _):
            for b in range(cw):
                rows, semg = bufs[b % 2]
                for kb in range(2):
                    @pl.when(k % 2 == kb)
                    def _do():
                        pltpu.make_async_copy(
                            tbl.at[srccs[kb].at[b]], rows, semg).wait()
                        pltpu.sync_copy(rows, acc.at[dstcs[kb].at[b]],
                                        add=True)
                        if b == cw - 2:
                            @pl.when(k + 1 < nchunk)
                            def _w():
                                wait_idx(1 - kb)
                        if b + 2 < cw:
                            pltpu.async_copy(tbl.at[srccs[kb].at[b + 2]],
                                             rows, semg)
                        else:
                            @pl.when(k + 1 < nchunk)
                            def _x():
                                pltpu.async_copy(
                                    tbl.at[srccs[1 - kb].at[b + 2 - cw]],
                                    rows, semg)

            @pl.when(k + 2 < nchunk)
            def _pf():
                for kb in range(2):
                    @pl.when(k % 2 == kb)
                    def _pf2():
                        load_idx_async(k + 2, kb)
            return _
        lax.fori_loop(0, nchunk, chunk, 0)

        plsc.subcore_barrier()
        pltpu.sync_copy(acc.at[pl.ds(r0, ROWS_PT)],
                        out.at[c, pl.ds(r0, ROWS_PT)])

    def call(tbl, src2d, dst2d):
        f = pl.kernel(
            body,
            out_type=jax.ShapeDtypeStruct((NC, NP, tw), BF),
            mesh=_mesh(),
            scratch_types=[
                pltpu.VMEM((cw, win), jnp.int32),
                pltpu.VMEM((cw, win), jnp.int32),
                pltpu.VMEM((cw, win), jnp.int32),
                pltpu.VMEM((cw, win), jnp.int32),
                pltpu.VMEM((win, tw), BF),
                pltpu.VMEM((win, tw), BF),
                pltpu.SemaphoreType.DMA,
                pltpu.SemaphoreType.DMA,
                pltpu.SemaphoreType.DMA,
                pltpu.VMEM_SHARED((NP, tw), BF),
            ],
            compiler_params=_SC_PARAMS,
        )
        return f(tbl, src2d, dst2d)

    return call


_spmm = _make_spmm(WIN, D, CW)
_spmm2 = _make_spmm(128, 2 * D, 8)


def _prep_body(cnt_ref, f1_ref, f2_ref, sc1_ref, sc2_ref, inv_ref):
    cnt = cnt_ref[0] + cnt_ref[1]
    inv = lax.rsqrt(jnp.maximum(cnt, 1.0))
    inv_ref[...] = inv
    sc1_ref[...] = (f1_ref[...] * inv[:, 0:1]).astype(BF)
    sc2_ref[...] = (f2_ref[...] * inv[:, 2:3]).astype(BF)


def _prep(cnts, f1p, f2p):
    grid = NP // BN
    return pl.pallas_call(
        _prep_body,
        grid=(grid,),
        in_specs=[
            pl.BlockSpec((NC, BN, DEG8), lambda i: (0, i, 0)),
            pl.BlockSpec((BN, D), lambda i: (i, 0)),
            pl.BlockSpec((BN, D), lambda i: (i, 0)),
        ],
        out_specs=[
            pl.BlockSpec((BN, D), lambda i: (i, 0)),
            pl.BlockSpec((BN, D), lambda i: (i, 0)),
            pl.BlockSpec((BN, DEG8), lambda i: (i, 0)),
        ],
        out_shape=[
            jax.ShapeDtypeStruct((NP, D), BF),
            jax.ShapeDtypeStruct((NP, D), BF),
            jax.ShapeDtypeStruct((NP, DEG8), jnp.float32),
        ],
    )(cnts, f1p, f2p)


def _mid_body(s1_ref, s2_ref, inv_ref, w11_ref, b11_ref, w21_ref, b21_ref,
              w12_ref, w22_ref, y1_ref, y2_ref):
    inv = inv_ref[...]
    for g, (s_ref, y_ref) in enumerate(((s1_ref, y1_ref), (s2_ref, y2_ref))):
        inv_in = inv[:, 2 * g + 1:2 * g + 2]
        inv_out = inv[:, 2 * g:2 * g + 1]
        S = (s_ref[0].astype(jnp.float32)
             + s_ref[1].astype(jnp.float32)) * inv_in
        ys = []
        for (w1_ref, b1_ref, w2_ref) in (
                (w11_ref, b11_ref, w12_ref),
                (w21_ref, b21_ref, w22_ref)):
            X = jnp.maximum(
                jnp.dot(S, w1_ref[...],
                        preferred_element_type=jnp.float32) + b1_ref[...],
                0.0)
            ys.append((jnp.dot(
                X, w2_ref[...],
                preferred_element_type=jnp.float32) * inv_out).astype(BF))
        y_ref[...] = jnp.concatenate(ys, axis=1)


def _mid(S1, S2, invs, W11, b11, W21, b21, W12, W22):
    grid = NP // BN
    full = lambda shape: pl.BlockSpec(shape, lambda i: tuple(0 for _ in shape))
    return pl.pallas_call(
        _mid_body,
        grid=(grid,),
        in_specs=[
            pl.BlockSpec((NC, BN, D), lambda i: (0, i, 0)),
            pl.BlockSpec((NC, BN, D), lambda i: (0, i, 0)),
            pl.BlockSpec((BN, DEG8), lambda i: (i, 0)),
            full((D, 2 * D)), full((1, 2 * D)),
            full((D, 2 * D)), full((1, 2 * D)),
            full((2 * D, D)), full((2 * D, D)),
        ],
        out_specs=[pl.BlockSpec((BN, 2 * D), lambda i: (i, 0))] * 2,
        out_shape=[jax.ShapeDtypeStruct((NP, 2 * D), BF)] * 2,
    )(S1, S2, invs, W11, b11, W21, b21, W12, W22)


def _fin_body(t1_ref, t2_ref, inv_ref,
              b12_ref, b22_ref, p11w_ref, p11b_ref, p12w_ref, p12b_ref,
              p21w_ref, p21b_ref, p22w_ref, p22b_ref,
              z1_ref, z2_ref, z1b_ref, z2b_ref):
    inv = inv_ref[...]
    jobs = (
        (t1_ref, 0, 0, b12_ref, p11w_ref, p11b_ref, p12w_ref, p12b_ref,
         z1_ref),
        (t2_ref, 1, 0, b12_ref, p11w_ref, p11b_ref, p12w_ref, p12b_ref,
         z2_ref),
        (t1_ref, 0, 1, b22_ref, p21w_ref, p21b_ref, p22w_ref, p22b_ref,
         z1b_ref),
        (t2_ref, 1, 1, b22_ref, p21w_ref, p21b_ref, p22w_ref, p22b_ref,
         z2b_ref),
    )
    for (t_ref, g, e, b2_ref, p1w_ref, p1b_ref, p2w_ref, p2b_ref,
         z_ref) in jobs:
        t = (t_ref[0, :, e * D:(e + 1) * D].astype(jnp.float32)
             + t_ref[1, :, e * D:(e + 1) * D].astype(jnp.float32))
        x = jnp.maximum(
            t * inv[:, 2 * g + 1:2 * g + 2] + b2_ref[...], 0.0)
        u = jnp.dot(x, p1w_ref[...],
                    preferred_element_type=jnp.float32) + p1b_ref[...]
        u = jnp.where(u > 0.0, u, jnp.exp(u) - 1.0)
        z_ref[...] = jnp.dot(u, p2w_ref[...],
                             preferred_element_type=jnp.float32) + p2b_ref[...]


def _fin(T1, T2, invs, b12, b22,
         p11w, p11b, p12w, p12b, p21w, p21b, p22w, p22b):
    grid = NP // BN
    full = lambda shape: pl.BlockSpec(shape, lambda i: tuple(0 for _ in shape))
    return pl.pallas_call(
        _fin_body,
        grid=(grid,),
        in_specs=[
            pl.BlockSpec((NC, BN, 2 * D), lambda i: (0, i, 0)),
            pl.BlockSpec((NC, BN, 2 * D), lambda i: (0, i, 0)),
            pl.BlockSpec((BN, DEG8), lambda i: (i, 0)),
            full((1, D)), full((1, D)),
            full((D, D)), full((1, D)), full((D, D)), full((1, D)),
            full((D, D)), full((1, D)), full((D, D)), full((1, D)),
        ],
        out_specs=[pl.BlockSpec((BN, D), lambda i: (i, 0))] * 4,
        out_shape=[jax.ShapeDtypeStruct((NP, D), jnp.float32)] * 4,
    )(T1, T2, invs, b12, b22,
      p11w, p11b, p12w, p12b, p21w, p21b, p22w, p22b)


def kernel(feat1, feat2, edge_index1, edge_index2,
           e1_W1, e1_b1, e1_W2, e1_b2, e1_p1W, e1_p1b, e1_p2W, e1_p2b,
           e2_W1, e2_b1, e2_W2, e2_b2, e2_p1W, e2_p1b, e2_p2W, e2_p2b,
           epoch, threshold, split_size):
    del epoch, threshold, split_size

    npad = EPAD - E
    padv = (jnp.arange(npad, dtype=jnp.int32) % (NP - N)) + N

    def pad_idx(x):
        return jnp.concatenate([x.astype(jnp.int32), padv])

    s1f = pad_idx(edge_index1[0])
    d1f = pad_idx(edge_index1[1])
    s2f = pad_idx(edge_index2[0])
    d2f = pad_idx(edge_index2[1])
    s1 = s1f.reshape(EPAD // WIN, WIN)
    d1 = d1f.reshape(EPAD // WIN, WIN)
    s2 = s2f.reshape(EPAD // WIN, WIN)
    d2 = d2f.reshape(EPAD // WIN, WIN)
    s1n = s1f.reshape(EPAD // 128, 128)
    d1n = d1f.reshape(EPAD // 128, 128)
    s2n = s2f.reshape(EPAD // 128, 128)
    d2n = d2f.reshape(EPAD // 128, 128)

    idx4 = jnp.stack([s1f * DEG8, d1f * DEG8 + 1,
                      s2f * DEG8 + 2, d2f * DEG8 + 3])
    cnts = _degrees(idx4).reshape(NC, NP, DEG8)

    f1p = jnp.pad(feat1, ((0, NP - N), (0, 0)))
    f2p = jnp.pad(feat2, ((0, NP - N), (0, 0)))
    featsc1, featsc2, invs = _prep(cnts, f1p, f2p)

    S1 = _spmm(featsc1, s1, d1)
    S2 = _spmm(featsc2, s2, d2)

    y1, y2 = _mid(
        S1, S2, invs,
        e1_W1, e1_b1.reshape(1, -1), e2_W1, e2_b1.reshape(1, -1),
        e1_W2, e2_W2)

    T1 = _spmm2(y1, s1n, d1n)
    T2 = _spmm2(y2, s2n, d2n)

    z1, z2, z1_, z2_ = _fin(
        T1, T2, invs,
        e1_b2.reshape(1, -1), e2_b2.reshape(1, -1),
        e1_p1W, e1_p1b.reshape(1, -1), e1_p2W, e1_p2b.reshape(1, -1),
        e2_p1W, e2_p1b.reshape(1, -1), e2_p2W, e2_p2b.reshape(1, -1))

    return (z1[:N], z2[:N], z1_[:N], z2_[:N])


# speedup vs baseline: 1.0921x; 1.0601x over previous
import jax
import jax.numpy as jnp
from jax import lax
from jax.experimental import pallas as pl
from jax.experimental.pallas import tpu as pltpu
from jax.experimental.pallas import tpu_sc as plsc

N = 10000
NP = 10240
E = 320000
D = 128
NC = 2
NS = 16
WIN = 256
EPAD = 327680
WPT = EPAD // (NC * NS * WIN)
ROWS_PT = NP // NS
CW = 8
NCHUNK = WPT // CW
BN = 512
DEG8 = 8
DBW = 2048
BF = jnp.bfloat16


def _mesh():
    return plsc.VectorSubcoreMesh(core_axis_name="c", subcore_axis_name="s",
                                  num_cores=NC, num_subcores=NS)


_SC_PARAMS = pltpu.CompilerParams(use_tc_tiling_on_sc=False)


def _deg_body(idx4, out, idxb, ones, zwin, sem, acc):
    c = lax.axis_index("c")
    s = lax.axis_index("s")

    def of(i, _):
        ones[pl.ds(i * 16, 16)] = jnp.ones((16,), jnp.float32)
        return _
    lax.fori_loop(0, DBW // 16, of, 0)

    def zf(i, _):
        zwin[pl.ds(i * 16, 16)] = jnp.zeros((16,), jnp.float32)
        return _
    lax.fori_loop(0, 80, zf, 0)
    zchunk = NP * DEG8 // NS
    zbase = s * zchunk
    for k in range(zchunk // 1280):
        pltpu.sync_copy(zwin, acc.at[pl.ds(zbase + k * 1280, 1280)])
    plsc.subcore_barrier()

    bw = DBW
    epw = EPAD // (NC * NS)
    eb = (c * NS + s) * epw
    nbig = epw // bw
    for a in range(4):
        pltpu.sync_copy(idx4.at[a, pl.ds(eb, epw)], idxb.at[a])
        for w in range(nbig):
            pltpu.async_copy(
                ones, acc.at[idxb.at[a, pl.ds(w * bw, bw)]],
                sem, add=True)
    for a in range(4):
        for w in range(nbig):
            pltpu.make_async_copy(
                ones, acc.at[idxb.at[0, pl.ds(0, bw)]], sem).wait()

    plsc.subcore_barrier()
    pltpu.sync_copy(acc.at[pl.ds(zbase, zchunk)],
                    out.at[c, pl.ds(zbase, zchunk)])


def _degrees(idx4):
    f = pl.kernel(
        _deg_body,
        out_type=jax.ShapeDtypeStruct((NC, NP * DEG8), jnp.float32),
        mesh=_mesh(),
        scratch_types=[
            pltpu.VMEM((4, EPAD // (NC * NS)), jnp.int32),
            pltpu.VMEM((DBW,), jnp.float32),
            pltpu.VMEM((1280,), jnp.float32),
            pltpu.SemaphoreType.DMA,
            pltpu.VMEM_SHARED((NP * DEG8,), jnp.float32),
        ],
        compiler_params=_SC_PARAMS,
    )
    return f(idx4)


def _make_spmm(win, tw, cw):
    wpt = EPAD // (NC * NS * win)
    nchunk = wpt // cw

    def body(tbl, src2d, dst2d, out, srcc0, srcc1, dstc0, dstc1,
             rows0, rows1, semg0, semg1, semi, acc):
        c = lax.axis_index("c")
        s = lax.axis_index("s")

        def zrow(i, _):
            for j in range(tw // 32):
                rows0[i, pl.ds(j * 32, 32)] = jnp.zeros((32,), BF)
            return _
        lax.fori_loop(0, win, zrow, 0)
        r0 = s * ROWS_PT
        zch = min(win, 128)
        for k in range(ROWS_PT // zch):
            pltpu.sync_copy(rows0.at[pl.ds(0, zch)],
                            acc.at[pl.ds(r0 + k * zch, zch)])
        plsc.subcore_barrier()

        eb = (c * NS + s) * wpt
        srccs = (srcc0, srcc1)
        dstcs = (dstc0, dstc1)

        def load_idx_async(k, kb):
            pltpu.async_copy(src2d.at[pl.ds(eb + k * cw, cw)], srccs[kb],
                             semi)
            pltpu.async_copy(dst2d.at[pl.ds(eb + k * cw, cw)], dstcs[kb],
                             semi)

        def wait_idx(kb):
            pltpu.make_async_copy(src2d.at[pl.ds(eb, cw)], srccs[kb],
                                  semi).wait()
            pltpu.make_async_copy(dst2d.at[pl.ds(eb, cw)], dstcs[kb],
                                  semi).wait()

        pltpu.sync_copy(src2d.at[pl.ds(eb, cw)], srcc0)
        pltpu.sync_copy(dst2d.at[pl.ds(eb, cw)], dstc0)
        load_idx_async(1, 1)

        pltpu.async_copy(tbl.at[srcc0.at[0]], rows0, semg0)
        pltpu.async_copy(tbl.at[srcc0.at[1]], rows1, semg1)

        bufs = ((rows0, semg0), (rows1, semg1))

        def chunk(k, _):
            for b in range(cw):
                rows, semg = bufs[b % 2]
                for kb in range(2):
                    @pl.when(k % 2 == kb)
                    def _do():
                        pltpu.make_async_copy(
                            tbl.at[srccs[kb].at[b]], rows, semg).wait()
                        pltpu.sync_copy(rows, acc.at[dstcs[kb].at[b]],
                                        add=True)
                        if b == cw - 2:
                            @pl.when(k + 1 < nchunk)
                            def _w():
                                wait_idx(1 - kb)
                        if b + 2 < cw:
                            pltpu.async_copy(tbl.at[srccs[kb].at[b + 2]],
                                             rows, semg)
                        else:
                            @pl.when(k + 1 < nchunk)
                            def _x():
                                pltpu.async_copy(
                                    tbl.at[srccs[1 - kb].at[b + 2 - cw]],
                                    rows, semg)

            @pl.when(k + 2 < nchunk)
            def _pf():
                for kb in range(2):
                    @pl.when(k % 2 == kb)
                    def _pf2():
                        load_idx_async(k + 2, kb)
            return _
        lax.fori_loop(0, nchunk, chunk, 0)

        plsc.subcore_barrier()
        pltpu.sync_copy(acc.at[pl.ds(r0, ROWS_PT)],
                        out.at[c, pl.ds(r0, ROWS_PT)])

    def call(tbl, src2d, dst2d):
        f = pl.kernel(
            body,
            out_type=jax.ShapeDtypeStruct((NC, NP, tw), BF),
            mesh=_mesh(),
            scratch_types=[
                pltpu.VMEM((cw, win), jnp.int32),
                pltpu.VMEM((cw, win), jnp.int32),
                pltpu.VMEM((cw, win), jnp.int32),
                pltpu.VMEM((cw, win), jnp.int32),
                pltpu.VMEM((win, tw), BF),
                pltpu.VMEM((win, tw), BF),
                pltpu.SemaphoreType.DMA,
                pltpu.SemaphoreType.DMA,
                pltpu.SemaphoreType.DMA,
                pltpu.VMEM_SHARED((NP, tw), BF),
            ],
            compiler_params=_SC_PARAMS,
        )
        return f(tbl, src2d, dst2d)

    return call


_spmm = _make_spmm(WIN, D, CW)
_spmm2 = _make_spmm(160, 2 * D, 8)


def _prep_body(cnt_ref, f1_ref, f2_ref, sc1_ref, sc2_ref, inv_ref):
    cnt = cnt_ref[0] + cnt_ref[1]
    inv = lax.rsqrt(jnp.maximum(cnt, 1.0))
    inv_ref[...] = inv
    sc1_ref[...] = (f1_ref[...] * inv[:, 0:1]).astype(BF)
    sc2_ref[...] = (f2_ref[...] * inv[:, 2:3]).astype(BF)


def _prep(cnts, f1p, f2p):
    grid = NP // BN
    return pl.pallas_call(
        _prep_body,
        grid=(grid,),
        in_specs=[
            pl.BlockSpec((NC, BN, DEG8), lambda i: (0, i, 0)),
            pl.BlockSpec((BN, D), lambda i: (i, 0)),
            pl.BlockSpec((BN, D), lambda i: (i, 0)),
        ],
        out_specs=[
            pl.BlockSpec((BN, D), lambda i: (i, 0)),
            pl.BlockSpec((BN, D), lambda i: (i, 0)),
            pl.BlockSpec((BN, DEG8), lambda i: (i, 0)),
        ],
        out_shape=[
            jax.ShapeDtypeStruct((NP, D), BF),
            jax.ShapeDtypeStruct((NP, D), BF),
            jax.ShapeDtypeStruct((NP, DEG8), jnp.float32),
        ],
    )(cnts, f1p, f2p)


def _mid_body(s1_ref, s2_ref, inv_ref, w11_ref, b11_ref, w21_ref, b21_ref,
              w12_ref, w22_ref, y1_ref, y2_ref):
    inv = inv_ref[...]
    for g, (s_ref, y_ref) in enumerate(((s1_ref, y1_ref), (s2_ref, y2_ref))):
        inv_in = inv[:, 2 * g + 1:2 * g + 2]
        inv_out = inv[:, 2 * g:2 * g + 1]
        S = (s_ref[0].astype(jnp.float32)
             + s_ref[1].astype(jnp.float32)) * inv_in
        ys = []
        for (w1_ref, b1_ref, w2_ref) in (
                (w11_ref, b11_ref, w12_ref),
                (w21_ref, b21_ref, w22_ref)):
            X = jnp.maximum(
                jnp.dot(S, w1_ref[...],
                        preferred_element_type=jnp.float32) + b1_ref[...],
                0.0)
            ys.append((jnp.dot(
                X, w2_ref[...],
                preferred_element_type=jnp.float32) * inv_out).astype(BF))
        y_ref[...] = jnp.concatenate(ys, axis=1)


def _mid(S1, S2, invs, W11, b11, W21, b21, W12, W22):
    grid = NP // BN
    full = lambda shape: pl.BlockSpec(shape, lambda i: tuple(0 for _ in shape))
    return pl.pallas_call(
        _mid_body,
        grid=(grid,),
        in_specs=[
            pl.BlockSpec((NC, BN, D), lambda i: (0, i, 0)),
            pl.BlockSpec((NC, BN, D), lambda i: (0, i, 0)),
            pl.BlockSpec((BN, DEG8), lambda i: (i, 0)),
            full((D, 2 * D)), full((1, 2 * D)),
            full((D, 2 * D)), full((1, 2 * D)),
            full((2 * D, D)), full((2 * D, D)),
        ],
        out_specs=[pl.BlockSpec((BN, 2 * D), lambda i: (i, 0))] * 2,
        out_shape=[jax.ShapeDtypeStruct((NP, 2 * D), BF)] * 2,
    )(S1, S2, invs, W11, b11, W21, b21, W12, W22)


def _fin_body(t1_ref, t2_ref, inv_ref,
              b12_ref, b22_ref, p11w_ref, p11b_ref, p12w_ref, p12b_ref,
              p21w_ref, p21b_ref, p22w_ref, p22b_ref,
              z1_ref, z2_ref, z1b_ref, z2b_ref):
    inv = inv_ref[...]
    jobs = (
        (t1_ref, 0, 0, b12_ref, p11w_ref, p11b_ref, p12w_ref, p12b_ref,
         z1_ref),
        (t2_ref, 1, 0, b12_ref, p11w_ref, p11b_ref, p12w_ref, p12b_ref,
         z2_ref),
        (t1_ref, 0, 1, b22_ref, p21w_ref, p21b_ref, p22w_ref, p22b_ref,
         z1b_ref),
        (t2_ref, 1, 1, b22_ref, p21w_ref, p21b_ref, p22w_ref, p22b_ref,
         z2b_ref),
    )
    for (t_ref, g, e, b2_ref, p1w_ref, p1b_ref, p2w_ref, p2b_ref,
         z_ref) in jobs:
        t = (t_ref[0, :, e * D:(e + 1) * D].astype(jnp.float32)
             + t_ref[1, :, e * D:(e + 1) * D].astype(jnp.float32))
        x = jnp.maximum(
            t * inv[:, 2 * g + 1:2 * g + 2] + b2_ref[...], 0.0)
        u = jnp.dot(x, p1w_ref[...],
                    preferred_element_type=jnp.float32) + p1b_ref[...]
        u = jnp.where(u > 0.0, u, jnp.exp(u) - 1.0)
        z_ref[...] = jnp.dot(u, p2w_ref[...],
                             preferred_element_type=jnp.float32) + p2b_ref[...]


def _fin(T1, T2, invs, b12, b22,
         p11w, p11b, p12w, p12b, p21w, p21b, p22w, p22b):
    grid = NP // BN
    full = lambda shape: pl.BlockSpec(shape, lambda i: tuple(0 for _ in shape))
    return pl.pallas_call(
        _fin_body,
        grid=(grid,),
        in_specs=[
            pl.BlockSpec((NC, BN, 2 * D), lambda i: (0, i, 0)),
            pl.BlockSpec((NC, BN, 2 * D), lambda i: (0, i, 0)),
            pl.BlockSpec((BN, DEG8), lambda i: (i, 0)),
            full((1, D)), full((1, D)),
            full((D, D)), full((1, D)), full((D, D)), full((1, D)),
            full((D, D)), full((1, D)), full((D, D)), full((1, D)),
        ],
        out_specs=[pl.BlockSpec((BN, D), lambda i: (i, 0))] * 4,
        out_shape=[jax.ShapeDtypeStruct((NP, D), jnp.float32)] * 4,
    )(T1, T2, invs, b12, b22,
      p11w, p11b, p12w, p12b, p21w, p21b, p22w, p22b)


def kernel(feat1, feat2, edge_index1, edge_index2,
           e1_W1, e1_b1, e1_W2, e1_b2, e1_p1W, e1_p1b, e1_p2W, e1_p2b,
           e2_W1, e2_b1, e2_W2, e2_b2, e2_p1W, e2_p1b, e2_p2W, e2_p2b,
           epoch, threshold, split_size):
    del epoch, threshold, split_size

    npad = EPAD - E
    padv = (jnp.arange(npad, dtype=jnp.int32) % (NP - N)) + N

    def pad_idx(x):
        return jnp.concatenate([x.astype(jnp.int32), padv])

    s1f = pad_idx(edge_index1[0])
    d1f = pad_idx(edge_index1[1])
    s2f = pad_idx(edge_index2[0])
    d2f = pad_idx(edge_index2[1])
    s1 = s1f.reshape(EPAD // WIN, WIN)
    d1 = d1f.reshape(EPAD // WIN, WIN)
    s2 = s2f.reshape(EPAD // WIN, WIN)
    d2 = d2f.reshape(EPAD // WIN, WIN)
    s1n = s1f.reshape(EPAD // 160, 160)
    d1n = d1f.reshape(EPAD // 160, 160)
    s2n = s2f.reshape(EPAD // 160, 160)
    d2n = d2f.reshape(EPAD // 160, 160)

    idx4 = jnp.stack([s1f * DEG8, d1f * DEG8 + 1,
                      s2f * DEG8 + 2, d2f * DEG8 + 3])
    cnts = _degrees(idx4).reshape(NC, NP, DEG8)

    f1p = jnp.pad(feat1, ((0, NP - N), (0, 0)))
    f2p = jnp.pad(feat2, ((0, NP - N), (0, 0)))
    featsc1, featsc2, invs = _prep(cnts, f1p, f2p)

    S1 = _spmm(featsc1, s1, d1)
    S2 = _spmm(featsc2, s2, d2)

    y1, y2 = _mid(
        S1, S2, invs,
        e1_W1, e1_b1.reshape(1, -1), e2_W1, e2_b1.reshape(1, -1),
        e1_W2, e2_W2)

    T1 = _spmm2(y1, s1n, d1n)
    T2 = _spmm2(y2, s2n, d2n)

    z1, z2, z1_, z2_ = _fin(
        T1, T2, invs,
        e1_b2.reshape(1, -1), e2_b2.reshape(1, -1),
        e1_p1W, e1_p1b.reshape(1, -1), e1_p2W, e1_p2b.reshape(1, -1),
        e2_p1W, e2_p1b.reshape(1, -1), e2_p2W, e2_p2b.reshape(1, -1))

    return (z1[:N], z2[:N], z1_[:N], z2_[:N])


# speedup vs baseline: 1.1271x; 1.0321x over previous
import jax
import jax.numpy as jnp
from jax import lax
from jax.experimental import pallas as pl
from jax.experimental.pallas import tpu as pltpu
from jax.experimental.pallas import tpu_sc as plsc

N = 10000
NP = 10240
E = 320000
D = 128
NC = 2
NS = 16
WIN = 256
EPAD = 327680
WPT = EPAD // (NC * NS * WIN)
ROWS_PT = NP // NS
CW = 8
NCHUNK = WPT // CW
BN = 1024
DEG8 = 8
DBW = 2048
BF = jnp.bfloat16


def _mesh():
    return plsc.VectorSubcoreMesh(core_axis_name="c", subcore_axis_name="s",
                                  num_cores=NC, num_subcores=NS)


_SC_PARAMS = pltpu.CompilerParams(use_tc_tiling_on_sc=False)


def _deg_body(idx4, out, idxb, ones, zwin, sem, acc):
    c = lax.axis_index("c")
    s = lax.axis_index("s")

    def of(i, _):
        ones[pl.ds(i * 16, 16)] = jnp.ones((16,), jnp.float32)
        return _
    lax.fori_loop(0, DBW // 16, of, 0)

    def zf(i, _):
        zwin[pl.ds(i * 16, 16)] = jnp.zeros((16,), jnp.float32)
        return _
    lax.fori_loop(0, 80, zf, 0)
    zchunk = NP * DEG8 // NS
    zbase = s * zchunk
    for k in range(zchunk // 1280):
        pltpu.sync_copy(zwin, acc.at[pl.ds(zbase + k * 1280, 1280)])
    plsc.subcore_barrier()

    bw = DBW
    epw = EPAD // (NC * NS)
    eb = (c * NS + s) * epw
    nbig = epw // bw
    for a in range(4):
        pltpu.sync_copy(idx4.at[a, pl.ds(eb, epw)], idxb.at[a])
        for w in range(nbig):
            pltpu.async_copy(
                ones, acc.at[idxb.at[a, pl.ds(w * bw, bw)]],
                sem, add=True)
    for a in range(4):
        for w in range(nbig):
            pltpu.make_async_copy(
                ones, acc.at[idxb.at[0, pl.ds(0, bw)]], sem).wait()

    plsc.subcore_barrier()
    pltpu.sync_copy(acc.at[pl.ds(zbase, zchunk)],
                    out.at[c, pl.ds(zbase, zchunk)])


def _degrees(idx4):
    f = pl.kernel(
        _deg_body,
        out_type=jax.ShapeDtypeStruct((NC, NP * DEG8), jnp.float32),
        mesh=_mesh(),
        scratch_types=[
            pltpu.VMEM((4, EPAD // (NC * NS)), jnp.int32),
            pltpu.VMEM((DBW,), jnp.float32),
            pltpu.VMEM((1280,), jnp.float32),
            pltpu.SemaphoreType.DMA,
            pltpu.VMEM_SHARED((NP * DEG8,), jnp.float32),
        ],
        compiler_params=_SC_PARAMS,
    )
    return f(idx4)


def _make_spmm(win, tw, cw):
    wpt = EPAD // (NC * NS * win)
    nchunk = wpt // cw

    def body(tbl, src2d, dst2d, out, srcc0, srcc1, dstc0, dstc1,
             rows0, rows1, semg0, semg1, semi, acc):
        c = lax.axis_index("c")
        s = lax.axis_index("s")

        def zrow(i, _):
            for j in range(tw // 32):
                rows0[i, pl.ds(j * 32, 32)] = jnp.zeros((32,), BF)
            return _
        lax.fori_loop(0, win, zrow, 0)
        r0 = s * ROWS_PT
        zch = min(win, 128)
        for k in range(ROWS_PT // zch):
            pltpu.sync_copy(rows0.at[pl.ds(0, zch)],
                            acc.at[pl.ds(r0 + k * zch, zch)])
        plsc.subcore_barrier()

        eb = (c * NS + s) * wpt
        srccs = (srcc0, srcc1)
        dstcs = (dstc0, dstc1)

        def load_idx_async(k, kb):
            pltpu.async_copy(src2d.at[pl.ds(eb + k * cw, cw)], srccs[kb],
                             semi)
            pltpu.async_copy(dst2d.at[pl.ds(eb + k * cw, cw)], dstcs[kb],
                             semi)

        def wait_idx(kb):
            pltpu.make_async_copy(src2d.at[pl.ds(eb, cw)], srccs[kb],
                                  semi).wait()
            pltpu.make_async_copy(dst2d.at[pl.ds(eb, cw)], dstcs[kb],
                                  semi).wait()

        pltpu.sync_copy(src2d.at[pl.ds(eb, cw)], srcc0)
        pltpu.sync_copy(dst2d.at[pl.ds(eb, cw)], dstc0)
        load_idx_async(1, 1)

        pltpu.async_copy(tbl.at[srcc0.at[0]], rows0, semg0)
        pltpu.async_copy(tbl.at[srcc0.at[1]], rows1, semg1)

        bufs = ((rows0, semg0), (rows1, semg1))

        def chunk(k, _):
            for b in range(cw):
                rows, semg = bufs[b % 2]
                for kb in range(2):
                    @pl.when(k % 2 == kb)
                    def _do():
                        pltpu.make_async_copy(
                            tbl.at[srccs[kb].at[b]], rows, semg).wait()
                        pltpu.sync_copy(rows, acc.at[dstcs[kb].at[b]],
                                        add=True)
                        if b == cw - 2:
                            @pl.when(k + 1 < nchunk)
                            def _w():
                                wait_idx(1 - kb)
                        if b + 2 < cw:
                            pltpu.async_copy(tbl.at[srccs[kb].at[b + 2]],
                                             rows, semg)
                        else:
                            @pl.when(k + 1 < nchunk)
                            def _x():
                                pltpu.async_copy(
                                    tbl.at[srccs[1 - kb].at[b + 2 - cw]],
                                    rows, semg)

            @pl.when(k + 2 < nchunk)
            def _pf():
                for kb in range(2):
                    @pl.when(k % 2 == kb)
                    def _pf2():
                        load_idx_async(k + 2, kb)
            return _
        lax.fori_loop(0, nchunk, chunk, 0)

        plsc.subcore_barrier()
        pltpu.sync_copy(acc.at[pl.ds(r0, ROWS_PT)],
                        out.at[c, pl.ds(r0, ROWS_PT)])

    def call(tbl, src2d, dst2d):
        f = pl.kernel(
            body,
            out_type=jax.ShapeDtypeStruct((NC, NP, tw), BF),
            mesh=_mesh(),
            scratch_types=[
                pltpu.VMEM((cw, win), jnp.int32),
                pltpu.VMEM((cw, win), jnp.int32),
                pltpu.VMEM((cw, win), jnp.int32),
                pltpu.VMEM((cw, win), jnp.int32),
                pltpu.VMEM((win, tw), BF),
                pltpu.VMEM((win, tw), BF),
                pltpu.SemaphoreType.DMA,
                pltpu.SemaphoreType.DMA,
                pltpu.SemaphoreType.DMA,
                pltpu.VMEM_SHARED((NP, tw), BF),
            ],
            compiler_params=_SC_PARAMS,
        )
        return f(tbl, src2d, dst2d)

    return call


_spmm = _make_spmm(WIN, D, CW)
_spmm2 = _make_spmm(160, 2 * D, 8)


def _prep_body(cnt_ref, f1_ref, f2_ref, sc1_ref, sc2_ref, inv_ref):
    cnt = cnt_ref[0] + cnt_ref[1]
    inv = lax.rsqrt(jnp.maximum(cnt, 1.0))
    inv_ref[...] = inv
    sc1_ref[...] = (f1_ref[...] * inv[:, 0:1]).astype(BF)
    sc2_ref[...] = (f2_ref[...] * inv[:, 2:3]).astype(BF)


def _prep(cnts, f1p, f2p):
    grid = NP // BN
    return pl.pallas_call(
        _prep_body,
        grid=(grid,),
        in_specs=[
            pl.BlockSpec((NC, BN, DEG8), lambda i: (0, i, 0)),
            pl.BlockSpec((BN, D), lambda i: (i, 0)),
            pl.BlockSpec((BN, D), lambda i: (i, 0)),
        ],
        out_specs=[
            pl.BlockSpec((BN, D), lambda i: (i, 0)),
            pl.BlockSpec((BN, D), lambda i: (i, 0)),
            pl.BlockSpec((BN, DEG8), lambda i: (i, 0)),
        ],
        out_shape=[
            jax.ShapeDtypeStruct((NP, D), BF),
            jax.ShapeDtypeStruct((NP, D), BF),
            jax.ShapeDtypeStruct((NP, DEG8), jnp.float32),
        ],
    )(cnts, f1p, f2p)


def _mid_body(s1_ref, s2_ref, inv_ref, w11_ref, b11_ref, w21_ref, b21_ref,
              w12_ref, w22_ref, y1_ref, y2_ref):
    inv = inv_ref[...]
    for g, (s_ref, y_ref) in enumerate(((s1_ref, y1_ref), (s2_ref, y2_ref))):
        inv_in = inv[:, 2 * g + 1:2 * g + 2]
        inv_out = inv[:, 2 * g:2 * g + 1]
        S = (s_ref[0].astype(jnp.float32)
             + s_ref[1].astype(jnp.float32)) * inv_in
        ys = []
        for (w1_ref, b1_ref, w2_ref) in (
                (w11_ref, b11_ref, w12_ref),
                (w21_ref, b21_ref, w22_ref)):
            X = jnp.maximum(
                jnp.dot(S, w1_ref[...],
                        preferred_element_type=jnp.float32) + b1_ref[...],
                0.0)
            ys.append((jnp.dot(
                X, w2_ref[...],
                preferred_element_type=jnp.float32) * inv_out).astype(BF))
        y_ref[...] = jnp.concatenate(ys, axis=1)


def _mid(S1, S2, invs, W11, b11, W21, b21, W12, W22):
    grid = NP // BN
    full = lambda shape: pl.BlockSpec(shape, lambda i: tuple(0 for _ in shape))
    return pl.pallas_call(
        _mid_body,
        grid=(grid,),
        in_specs=[
            pl.BlockSpec((NC, BN, D), lambda i: (0, i, 0)),
            pl.BlockSpec((NC, BN, D), lambda i: (0, i, 0)),
            pl.BlockSpec((BN, DEG8), lambda i: (i, 0)),
            full((D, 2 * D)), full((1, 2 * D)),
            full((D, 2 * D)), full((1, 2 * D)),
            full((2 * D, D)), full((2 * D, D)),
        ],
        out_specs=[pl.BlockSpec((BN, 2 * D), lambda i: (i, 0))] * 2,
        out_shape=[jax.ShapeDtypeStruct((NP, 2 * D), BF)] * 2,
    )(S1, S2, invs, W11, b11, W21, b21, W12, W22)


def _fin_body(t1_ref, t2_ref, inv_ref,
              b12_ref, b22_ref, p11w_ref, p11b_ref, p12w_ref, p12b_ref,
              p21w_ref, p21b_ref, p22w_ref, p22b_ref,
              z1_ref, z2_ref, z1b_ref, z2b_ref):
    inv = inv_ref[...]
    jobs = (
        (t1_ref, 0, 0, b12_ref, p11w_ref, p11b_ref, p12w_ref, p12b_ref,
         z1_ref),
        (t2_ref, 1, 0, b12_ref, p11w_ref, p11b_ref, p12w_ref, p12b_ref,
         z2_ref),
        (t1_ref, 0, 1, b22_ref, p21w_ref, p21b_ref, p22w_ref, p22b_ref,
         z1b_ref),
        (t2_ref, 1, 1, b22_ref, p21w_ref, p21b_ref, p22w_ref, p22b_ref,
         z2b_ref),
    )
    for (t_ref, g, e, b2_ref, p1w_ref, p1b_ref, p2w_ref, p2b_ref,
         z_ref) in jobs:
        t = (t_ref[0, :, e * D:(e + 1) * D].astype(jnp.float32)
             + t_ref[1, :, e * D:(e + 1) * D].astype(jnp.float32))
        x = jnp.maximum(
            t * inv[:, 2 * g + 1:2 * g + 2] + b2_ref[...], 0.0)
        u = jnp.dot(x, p1w_ref[...],
                    preferred_element_type=jnp.float32) + p1b_ref[...]
        u = jnp.where(u > 0.0, u, jnp.exp(u) - 1.0)
        z_ref[...] = jnp.dot(u, p2w_ref[...],
                             preferred_element_type=jnp.float32) + p2b_ref[...]


def _fin(T1, T2, invs, b12, b22,
         p11w, p11b, p12w, p12b, p21w, p21b, p22w, p22b):
    grid = NP // BN
    full = lambda shape: pl.BlockSpec(shape, lambda i: tuple(0 for _ in shape))
    return pl.pallas_call(
        _fin_body,
        grid=(grid,),
        in_specs=[
            pl.BlockSpec((NC, BN, 2 * D), lambda i: (0, i, 0)),
            pl.BlockSpec((NC, BN, 2 * D), lambda i: (0, i, 0)),
            pl.BlockSpec((BN, DEG8), lambda i: (i, 0)),
            full((1, D)), full((1, D)),
            full((D, D)), full((1, D)), full((D, D)), full((1, D)),
            full((D, D)), full((1, D)), full((D, D)), full((1, D)),
        ],
        out_specs=[pl.BlockSpec((BN, D), lambda i: (i, 0))] * 4,
        out_shape=[jax.ShapeDtypeStruct((NP, D), jnp.float32)] * 4,
    )(T1, T2, invs, b12, b22,
      p11w, p11b, p12w, p12b, p21w, p21b, p22w, p22b)


def kernel(feat1, feat2, edge_index1, edge_index2,
           e1_W1, e1_b1, e1_W2, e1_b2, e1_p1W, e1_p1b, e1_p2W, e1_p2b,
           e2_W1, e2_b1, e2_W2, e2_b2, e2_p1W, e2_p1b, e2_p2W, e2_p2b,
           epoch, threshold, split_size):
    del epoch, threshold, split_size

    npad = EPAD - E
    padv = (jnp.arange(npad, dtype=jnp.int32) % (NP - N)) + N

    def pad_idx(x):
        return jnp.concatenate([x.astype(jnp.int32), padv])

    s1f = pad_idx(edge_index1[0])
    d1f = pad_idx(edge_index1[1])
    s2f = pad_idx(edge_index2[0])
    d2f = pad_idx(edge_index2[1])
    s1 = s1f.reshape(EPAD // WIN, WIN)
    d1 = d1f.reshape(EPAD // WIN, WIN)
    s2 = s2f.reshape(EPAD // WIN, WIN)
    d2 = d2f.reshape(EPAD // WIN, WIN)
    s1n = s1f.reshape(EPAD // 160, 160)
    d1n = d1f.reshape(EPAD // 160, 160)
    s2n = s2f.reshape(EPAD // 160, 160)
    d2n = d2f.reshape(EPAD // 160, 160)

    idx4 = jnp.stack([s1f * DEG8, d1f * DEG8 + 1,
                      s2f * DEG8 + 2, d2f * DEG8 + 3])
    cnts = _degrees(idx4).reshape(NC, NP, DEG8)

    f1p = jnp.pad(feat1, ((0, NP - N), (0, 0)))
    f2p = jnp.pad(feat2, ((0, NP - N), (0, 0)))
    featsc1, featsc2, invs = _prep(cnts, f1p, f2p)

    S1 = _spmm(featsc1, s1, d1)
    S2 = _spmm(featsc2, s2, d2)

    y1, y2 = _mid(
        S1, S2, invs,
        e1_W1, e1_b1.reshape(1, -1), e2_W1, e2_b1.reshape(1, -1),
        e1_W2, e2_W2)

    T1 = _spmm2(y1, s1n, d1n)
    T2 = _spmm2(y2, s2n, d2n)

    z1, z2, z1_, z2_ = _fin(
        T1, T2, invs,
        e1_b2.reshape(1, -1), e2_b2.reshape(1, -1),
        e1_p1W, e1_p1b.reshape(1, -1), e1_p2W, e1_p2b.reshape(1, -1),
        e2_p1W, e2_p1b.reshape(1, -1), e2_p2W, e2_p2b.reshape(1, -1))

    return (z1[:N], z2[:N], z1_[:N], z2_[:N])


# speedup vs baseline: 1.1327x; 1.0050x over previous
import jax
import jax.numpy as jnp
from jax import lax
from jax.experimental import pallas as pl
from jax.experimental.pallas import tpu as pltpu
from jax.experimental.pallas import tpu_sc as plsc

N = 10000
NP = 10240
E = 320000
D = 128
NC = 2
NS = 16
WIN = 256
EPAD = 327680
WPT = EPAD // (NC * NS * WIN)
ROWS_PT = NP // NS
CW = 8
NCHUNK = WPT // CW
BN = 2048
DEG8 = 8
DBW = 2048
BF = jnp.bfloat16


def _mesh():
    return plsc.VectorSubcoreMesh(core_axis_name="c", subcore_axis_name="s",
                                  num_cores=NC, num_subcores=NS)


_SC_PARAMS = pltpu.CompilerParams(use_tc_tiling_on_sc=False)


def _deg_body(idx4, out, idxb, ones, zwin, sem, acc):
    c = lax.axis_index("c")
    s = lax.axis_index("s")

    def of(i, _):
        ones[pl.ds(i * 16, 16)] = jnp.ones((16,), jnp.float32)
        return _
    lax.fori_loop(0, DBW // 16, of, 0)

    def zf(i, _):
        zwin[pl.ds(i * 16, 16)] = jnp.zeros((16,), jnp.float32)
        return _
    lax.fori_loop(0, 80, zf, 0)
    zchunk = NP * DEG8 // NS
    zbase = s * zchunk
    for k in range(zchunk // 1280):
        pltpu.sync_copy(zwin, acc.at[pl.ds(zbase + k * 1280, 1280)])
    plsc.subcore_barrier()

    bw = DBW
    epw = EPAD // (NC * NS)
    eb = (c * NS + s) * epw
    nbig = epw // bw
    for a in range(4):
        pltpu.sync_copy(idx4.at[a, pl.ds(eb, epw)], idxb.at[a])
        for w in range(nbig):
            pltpu.async_copy(
                ones, acc.at[idxb.at[a, pl.ds(w * bw, bw)]],
                sem, add=True)
    for a in range(4):
        for w in range(nbig):
            pltpu.make_async_copy(
                ones, acc.at[idxb.at[0, pl.ds(0, bw)]], sem).wait()

    plsc.subcore_barrier()
    pltpu.sync_copy(acc.at[pl.ds(zbase, zchunk)],
                    out.at[c, pl.ds(zbase, zchunk)])


def _degrees(idx4):
    f = pl.kernel(
        _deg_body,
        out_type=jax.ShapeDtypeStruct((NC, NP * DEG8), jnp.float32),
        mesh=_mesh(),
        scratch_types=[
            pltpu.VMEM((4, EPAD // (NC * NS)), jnp.int32),
            pltpu.VMEM((DBW,), jnp.float32),
            pltpu.VMEM((1280,), jnp.float32),
            pltpu.SemaphoreType.DMA,
            pltpu.VMEM_SHARED((NP * DEG8,), jnp.float32),
        ],
        compiler_params=_SC_PARAMS,
    )
    return f(idx4)


def _make_spmm(win, tw, cw):
    wpt = EPAD // (NC * NS * win)
    nchunk = wpt // cw

    def body(tbl, src2d, dst2d, out, srcc0, srcc1, dstc0, dstc1,
             rows0, rows1, semg0, semg1, semi, acc):
        c = lax.axis_index("c")
        s = lax.axis_index("s")

        def zrow(i, _):
            for j in range(tw // 32):
                rows0[i, pl.ds(j * 32, 32)] = jnp.zeros((32,), BF)
            return _
        lax.fori_loop(0, win, zrow, 0)
        r0 = s * ROWS_PT
        zch = min(win, 128)
        for k in range(ROWS_PT // zch):
            pltpu.sync_copy(rows0.at[pl.ds(0, zch)],
                            acc.at[pl.ds(r0 + k * zch, zch)])
        plsc.subcore_barrier()

        eb = (c * NS + s) * wpt
        srccs = (srcc0, srcc1)
        dstcs = (dstc0, dstc1)

        def load_idx_async(k, kb):
            pltpu.async_copy(src2d.at[pl.ds(eb + k * cw, cw)], srccs[kb],
                             semi)
            pltpu.async_copy(dst2d.at[pl.ds(eb + k * cw, cw)], dstcs[kb],
                             semi)

        def wait_idx(kb):
            pltpu.make_async_copy(src2d.at[pl.ds(eb, cw)], srccs[kb],
                                  semi).wait()
            pltpu.make_async_copy(dst2d.at[pl.ds(eb, cw)], dstcs[kb],
                                  semi).wait()

        pltpu.sync_copy(src2d.at[pl.ds(eb, cw)], srcc0)
        pltpu.sync_copy(dst2d.at[pl.ds(eb, cw)], dstc0)
        load_idx_async(1, 1)

        pltpu.async_copy(tbl.at[srcc0.at[0]], rows0, semg0)
        pltpu.async_copy(tbl.at[srcc0.at[1]], rows1, semg1)

        bufs = ((rows0, semg0), (rows1, semg1))

        def chunk(k, _):
            for b in range(cw):
                rows, semg = bufs[b % 2]
                for kb in range(2):
                    @pl.when(k % 2 == kb)
                    def _do():
                        pltpu.make_async_copy(
                            tbl.at[srccs[kb].at[b]], rows, semg).wait()
                        pltpu.sync_copy(rows, acc.at[dstcs[kb].at[b]],
                                        add=True)
                        if b == cw - 2:
                            @pl.when(k + 1 < nchunk)
                            def _w():
                                wait_idx(1 - kb)
                        if b + 2 < cw:
                            pltpu.async_copy(tbl.at[srccs[kb].at[b + 2]],
                                             rows, semg)
                        else:
                            @pl.when(k + 1 < nchunk)
                            def _x():
                                pltpu.async_copy(
                                    tbl.at[srccs[1 - kb].at[b + 2 - cw]],
                                    rows, semg)

            @pl.when(k + 2 < nchunk)
            def _pf():
                for kb in range(2):
                    @pl.when(k % 2 == kb)
                    def _pf2():
                        load_idx_async(k + 2, kb)
            return _
        lax.fori_loop(0, nchunk, chunk, 0)

        plsc.subcore_barrier()
        pltpu.sync_copy(acc.at[pl.ds(r0, ROWS_PT)],
                        out.at[c, pl.ds(r0, ROWS_PT)])

    def call(tbl, src2d, dst2d):
        f = pl.kernel(
            body,
            out_type=jax.ShapeDtypeStruct((NC, NP, tw), BF),
            mesh=_mesh(),
            scratch_types=[
                pltpu.VMEM((cw, win), jnp.int32),
                pltpu.VMEM((cw, win), jnp.int32),
                pltpu.VMEM((cw, win), jnp.int32),
                pltpu.VMEM((cw, win), jnp.int32),
                pltpu.VMEM((win, tw), BF),
                pltpu.VMEM((win, tw), BF),
                pltpu.SemaphoreType.DMA,
                pltpu.SemaphoreType.DMA,
                pltpu.SemaphoreType.DMA,
                pltpu.VMEM_SHARED((NP, tw), BF),
            ],
            compiler_params=_SC_PARAMS,
        )
        return f(tbl, src2d, dst2d)

    return call


_spmm = _make_spmm(WIN, D, CW)
_spmm2 = _make_spmm(160, 2 * D, 8)


def _prep_body(cnt_ref, f1_ref, f2_ref, sc1_ref, sc2_ref, inv_ref):
    cnt = cnt_ref[0] + cnt_ref[1]
    inv = lax.rsqrt(jnp.maximum(cnt, 1.0))
    inv_ref[...] = inv
    sc1_ref[...] = (f1_ref[...] * inv[:, 0:1]).astype(BF)
    sc2_ref[...] = (f2_ref[...] * inv[:, 2:3]).astype(BF)


def _prep(cnts, f1p, f2p):
    grid = NP // BN
    return pl.pallas_call(
        _prep_body,
        grid=(grid,),
        in_specs=[
            pl.BlockSpec((NC, BN, DEG8), lambda i: (0, i, 0)),
            pl.BlockSpec((BN, D), lambda i: (i, 0)),
            pl.BlockSpec((BN, D), lambda i: (i, 0)),
        ],
        out_specs=[
            pl.BlockSpec((BN, D), lambda i: (i, 0)),
            pl.BlockSpec((BN, D), lambda i: (i, 0)),
            pl.BlockSpec((BN, DEG8), lambda i: (i, 0)),
        ],
        out_shape=[
            jax.ShapeDtypeStruct((NP, D), BF),
            jax.ShapeDtypeStruct((NP, D), BF),
            jax.ShapeDtypeStruct((NP, DEG8), jnp.float32),
        ],
    )(cnts, f1p, f2p)


def _mid_body(s1_ref, s2_ref, inv_ref, w11_ref, b11_ref, w21_ref, b21_ref,
              w12_ref, w22_ref, y1_ref, y2_ref):
    inv = inv_ref[...]
    for g, (s_ref, y_ref) in enumerate(((s1_ref, y1_ref), (s2_ref, y2_ref))):
        inv_in = inv[:, 2 * g + 1:2 * g + 2]
        inv_out = inv[:, 2 * g:2 * g + 1]
        S = (s_ref[0].astype(jnp.float32)
             + s_ref[1].astype(jnp.float32)) * inv_in
        ys = []
        for (w1_ref, b1_ref, w2_ref) in (
                (w11_ref, b11_ref, w12_ref),
                (w21_ref, b21_ref, w22_ref)):
            X = jnp.maximum(
                jnp.dot(S, w1_ref[...],
                        preferred_element_type=jnp.float32) + b1_ref[...],
                0.0)
            ys.append((jnp.dot(
                X, w2_ref[...],
                preferred_element_type=jnp.float32) * inv_out).astype(BF))
        y_ref[...] = jnp.concatenate(ys, axis=1)


def _mid(S1, S2, invs, W11, b11, W21, b21, W12, W22):
    grid = NP // BN
    full = lambda shape: pl.BlockSpec(shape, lambda i: tuple(0 for _ in shape))
    return pl.pallas_call(
        _mid_body,
        grid=(grid,),
        in_specs=[
            pl.BlockSpec((NC, BN, D), lambda i: (0, i, 0)),
            pl.BlockSpec((NC, BN, D), lambda i: (0, i, 0)),
            pl.BlockSpec((BN, DEG8), lambda i: (i, 0)),
            full((D, 2 * D)), full((1, 2 * D)),
            full((D, 2 * D)), full((1, 2 * D)),
            full((2 * D, D)), full((2 * D, D)),
        ],
        out_specs=[pl.BlockSpec((BN, 2 * D), lambda i: (i, 0))] * 2,
        out_shape=[jax.ShapeDtypeStruct((NP, 2 * D), BF)] * 2,
    )(S1, S2, invs, W11, b11, W21, b21, W12, W22)


def _fin_body(t1_ref, t2_ref, inv_ref,
              b12_ref, b22_ref, p11w_ref, p11b_ref, p12w_ref, p12b_ref,
              p21w_ref, p21b_ref, p22w_ref, p22b_ref,
              z1_ref, z2_ref, z1b_ref, z2b_ref):
    inv = inv_ref[...]
    jobs = (
        (t1_ref, 0, 0, b12_ref, p11w_ref, p11b_ref, p12w_ref, p12b_ref,
         z1_ref),
        (t2_ref, 1, 0, b12_ref, p11w_ref, p11b_ref, p12w_ref, p12b_ref,
         z2_ref),
        (t1_ref, 0, 1, b22_ref, p21w_ref, p21b_ref, p22w_ref, p22b_ref,
         z1b_ref),
        (t2_ref, 1, 1, b22_ref, p21w_ref, p21b_ref, p22w_ref, p22b_ref,
         z2b_ref),
    )
    for (t_ref, g, e, b2_ref, p1w_ref, p1b_ref, p2w_ref, p2b_ref,
         z_ref) in jobs:
        t = (t_ref[0, :, e * D:(e + 1) * D].astype(jnp.float32)
             + t_ref[1, :, e * D:(e + 1) * D].astype(jnp.float32))
        x = jnp.maximum(
            t * inv[:, 2 * g + 1:2 * g + 2] + b2_ref[...], 0.0)
        u = jnp.dot(x, p1w_ref[...],
                    preferred_element_type=jnp.float32) + p1b_ref[...]
        u = jnp.where(u > 0.0, u, jnp.exp(u) - 1.0)
        z_ref[...] = jnp.dot(u, p2w_ref[...],
                             preferred_element_type=jnp.float32) + p2b_ref[...]


def _fin(T1, T2, invs, b12, b22,
         p11w, p11b, p12w, p12b, p21w, p21b, p22w, p22b):
    grid = NP // BN
    full = lambda shape: pl.BlockSpec(shape, lambda i: tuple(0 for _ in shape))
    return pl.pallas_call(
        _fin_body,
        grid=(grid,),
        in_specs=[
            pl.BlockSpec((NC, BN, 2 * D), lambda i: (0, i, 0)),
            pl.BlockSpec((NC, BN, 2 * D), lambda i: (0, i, 0)),
            pl.BlockSpec((BN, DEG8), lambda i: (i, 0)),
            full((1, D)), full((1, D)),
            full((D, D)), full((1, D)), full((D, D)), full((1, D)),
            full((D, D)), full((1, D)), full((D, D)), full((1, D)),
        ],
        out_specs=[pl.BlockSpec((BN, D), lambda i: (i, 0))] * 4,
        out_shape=[jax.ShapeDtypeStruct((NP, D), jnp.float32)] * 4,
    )(T1, T2, invs, b12, b22,
      p11w, p11b, p12w, p12b, p21w, p21b, p22w, p22b)


def kernel(feat1, feat2, edge_index1, edge_index2,
           e1_W1, e1_b1, e1_W2, e1_b2, e1_p1W, e1_p1b, e1_p2W, e1_p2b,
           e2_W1, e2_b1, e2_W2, e2_b2, e2_p1W, e2_p1b, e2_p2W, e2_p2b,
           epoch, threshold, split_size):
    del epoch, threshold, split_size

    npad = EPAD - E
    padv = (jnp.arange(npad, dtype=jnp.int32) % (NP - N)) + N

    def pad_idx(x):
        return jnp.concatenate([x.astype(jnp.int32), padv])

    s1f = pad_idx(edge_index1[0])
    d1f = pad_idx(edge_index1[1])
    s2f = pad_idx(edge_index2[0])
    d2f = pad_idx(edge_index2[1])
    s1 = s1f.reshape(EPAD // WIN, WIN)
    d1 = d1f.reshape(EPAD // WIN, WIN)
    s2 = s2f.reshape(EPAD // WIN, WIN)
    d2 = d2f.reshape(EPAD // WIN, WIN)
    s1n = s1f.reshape(EPAD // 160, 160)
    d1n = d1f.reshape(EPAD // 160, 160)
    s2n = s2f.reshape(EPAD // 160, 160)
    d2n = d2f.reshape(EPAD // 160, 160)

    idx4 = jnp.stack([s1f * DEG8, d1f * DEG8 + 1,
                      s2f * DEG8 + 2, d2f * DEG8 + 3])
    cnts = _degrees(idx4).reshape(NC, NP, DEG8)

    f1p = jnp.pad(feat1, ((0, NP - N), (0, 0)))
    f2p = jnp.pad(feat2, ((0, NP - N), (0, 0)))
    featsc1, featsc2, invs = _prep(cnts, f1p, f2p)

    S1 = _spmm(featsc1, s1, d1)
    S2 = _spmm(featsc2, s2, d2)

    y1, y2 = _mid(
        S1, S2, invs,
        e1_W1, e1_b1.reshape(1, -1), e2_W1, e2_b1.reshape(1, -1),
        e1_W2, e2_W2)

    T1 = _spmm2(y1, s1n, d1n)
    T2 = _spmm2(y2, s2n, d2n)

    z1, z2, z1_, z2_ = _fin(
        T1, T2, invs,
        e1_b2.reshape(1, -1), e2_b2.reshape(1, -1),
        e1_p1W, e1_p1b.reshape(1, -1), e1_p2W, e1_p2b.reshape(1, -1),
        e2_p1W, e2_p1b.reshape(1, -1), e2_p2W, e2_p2b.reshape(1, -1))

    return (z1[:N], z2[:N], z1_[:N], z2_[:N])
